# baseline (speedup 1.0000x reference)
import functools

import jax
import jax.numpy as jnp
from jax import lax
from jax.experimental import pallas as pl
from jax.experimental.pallas import tpu as pltpu
from jax.experimental.pallas import tpu_sc as plsc

_B, _H, _W, _C = 4, 112, 112, 192
_NB = _H * _W * _C
_N = _B * _NB
_OB = (2 * _H) * (2 * _W) * _C
_OUT_WORDS = _B * _OB

_TRASH = 16384
_CH = 15 * 131072 - _TRASH
_K = 5
_LAST = _OB - (_K - 1) * _CH
_NSUB = 16
_PT = _NB // _NSUB
_WIN = 1536
_NWIN = _PT // _WIN
_ZSH = _CH // _NSUB
_SSH = _CH // _NSUB
_LSH = _LAST // _NSUB
_NROUND = (_B * _K) // 2


_PRE_ROWS = 1024


def _detile(x):
    r = x.shape[0]
    m2 = x.reshape(r // 2, 2, _C)
    p0 = m2[:, 0, :]
    p1 = m2[:, 1, :]
    a0 = p0[:, :128]
    a1 = jnp.concatenate([p0[:, 128:], p1[:, :64]], axis=1)
    a2 = p1[:, 64:]
    return jnp.stack([a0, a1, a2], axis=1).reshape(r * 3 // 2, 128)


def _retile(x):
    r = x.shape[0]
    a = x.reshape(r // 3, 3, 128)
    b0 = jnp.concatenate([a[:, 0, :], a[:, 1, :64]], axis=1)
    b1 = jnp.concatenate([a[:, 1, 64:], a[:, 2, :]], axis=1)
    return jnp.stack([b0, b1], axis=1).reshape(r * 2 // 3, 192)


def _pre_body(mask_ref, upd_ref, dest_ref, uflat_ref):
    m = mask_ref[...]
    ch = lax.broadcasted_iota(jnp.int32, m.shape, 1)
    dest_ref[...] = _detile(m - m % _C + ch)
    uflat_ref[...] = _detile(upd_ref[...])


def _preprocess(mask, updates):
    m2 = mask.reshape(_B * _H * _W, _C)
    u2 = updates.reshape(_B * _H * _W, _C)
    nblk = _B * _H * _W // _PRE_ROWS
    obk = _PRE_ROWS * 3 // 2
    dest, uflat = pl.pallas_call(
        _pre_body,
        out_shape=(jax.ShapeDtypeStruct((_N // 128, 128), jnp.int32),
                   jax.ShapeDtypeStruct((_N // 128, 128), jnp.float32)),
        grid=(nblk,),
        in_specs=[pl.BlockSpec((_PRE_ROWS, _C), lambda i: (i, 0)),
                  pl.BlockSpec((_PRE_ROWS, _C), lambda i: (i, 0))],
        out_specs=(pl.BlockSpec((obk, 128), lambda i: (i, 0)),
                   pl.BlockSpec((obk, 128), lambda i: (i, 0))),
    )(m2, u2)
    return dest.reshape(_N), uflat.reshape(_N)


_POST_ROWS = 1024


def _post_body(flat_ref, out_ref):
    out_ref[...] = _retile(flat_ref[...])


def _postprocess(flat):
    orows = _B * (2 * _H) * (2 * _W)
    ibk = _POST_ROWS * 3 // 2
    out = pl.pallas_call(
        _post_body,
        out_shape=jax.ShapeDtypeStruct((orows, _C), jnp.float32),
        grid=(orows // _POST_ROWS,),
        in_specs=[pl.BlockSpec((ibk, 128), lambda i: (i, 0))],
        out_specs=pl.BlockSpec((_POST_ROWS, _C), lambda i: (i, 0)),
    )(flat.reshape(_OUT_WORDS // 128, 128))
    return out.reshape(_B, 2 * _H, 2 * _W, _C)


_mesh = plsc.VectorSubcoreMesh(core_axis_name="c", subcore_axis_name="s")


@functools.partial(
    pl.kernel,
    out_type=jax.ShapeDtypeStruct((_OUT_WORDS,), jnp.float32),
    mesh=_mesh,
    scratch_types=[
        pltpu.VMEM_SHARED((_CH + _TRASH,), jnp.float32),
        pltpu.VMEM((_WIN,), jnp.int32),
        pltpu.VMEM((_WIN,), jnp.int32),
        pltpu.VMEM((_WIN,), jnp.float32),
        pltpu.VMEM((_WIN,), jnp.float32),
        pltpu.VMEM((_WIN,), jnp.int32),
        pltpu.SemaphoreType.DMA,
        pltpu.SemaphoreType.DMA,
        pltpu.SemaphoreType.DMA,
        pltpu.SemaphoreType.DMA,
    ],
)
def _sc_scatter(dest_hbm, upd_hbm, zero_hbm, out_hbm, acc_sh, d0, d1, u0, u1,
                idx_v, sd0, sd1, su0, su1):
    core = lax.axis_index("c")
    sub = lax.axis_index("s")

    @pl.loop(0, _NROUND)
    def _(r):
        g = r * 2 + core
        b = g // _K
        k = g % _K
        ibase = b * _NB + sub * _PT
        cbase = k * _CH

        zoff = pl.multiple_of(sub * _ZSH, 8)
        pltpu.sync_copy(zero_hbm.at[pl.ds(zoff, _ZSH)],
                        acc_sh.at[pl.ds(zoff, _ZSH)])
        plsc.subcore_barrier()

        def start_load(w, db, ub, sdb, sub_):
            off = pl.multiple_of(ibase + w * _WIN, 8)
            pltpu.async_copy(dest_hbm.at[pl.ds(off, _WIN)], db, sdb)
            pltpu.async_copy(upd_hbm.at[pl.ds(off, _WIN)], ub, sub_)

        def wait_load(db, ub, sdb, sub_):
            pltpu.make_async_copy(dest_hbm.at[pl.ds(0, _WIN)], db, sdb).wait()
            pltpu.make_async_copy(upd_hbm.at[pl.ds(0, _WIN)], ub, sub_).wait()

        def process(db, ub):
            @pl.loop(0, _WIN // 16)
            def _(i):
                d = db[pl.ds(i * 16, 16)]
                rel = d - cbase
                inb = plsc.bitcast(rel, jnp.uint32) < jnp.uint32(_CH)
                tr = _CH + (d & (_TRASH - 1))
                idx_v[pl.ds(i * 16, 16)] = jnp.where(inb, rel, tr)

            pltpu.sync_copy(ub, acc_sh.at[idx_v], add=True)

        start_load(0, d0, u0, sd0, su0)
        start_load(1, d1, u1, sd1, su1)

        @pl.loop(0, _NWIN // 2 - 1)
        def _(j):
            w = j * 2
            wait_load(d0, u0, sd0, su0)
            process(d0, u0)
            start_load(w + 2, d0, u0, sd0, su0)
            wait_load(d1, u1, sd1, su1)
            process(d1, u1)
            start_load(w + 3, d1, u1, sd1, su1)

        wait_load(d0, u0, sd0, su0)
        process(d0, u0)
        wait_load(d1, u1, sd1, su1)
        process(d1, u1)
        plsc.subcore_barrier()

        @pl.when(k < _K - 1)
        def _():
            soff = pl.multiple_of(sub * _SSH, 8)
            obase = pl.multiple_of(b * _OB + k * _CH + sub * _SSH, 8)
            pltpu.sync_copy(acc_sh.at[pl.ds(soff, _SSH)],
                            out_hbm.at[pl.ds(obase, _SSH)])

        @pl.when(k == _K - 1)
        def _():
            soff = pl.multiple_of(sub * _LSH, 8)
            obase = pl.multiple_of(b * _OB + k * _CH + sub * _LSH, 8)
            pltpu.sync_copy(acc_sh.at[pl.ds(soff, _LSH)],
                            out_hbm.at[pl.ds(obase, _LSH)])

        plsc.subcore_barrier()


def kernel(updates, mask):
    mask = mask.astype(jnp.int32)
    dest, upd = _preprocess(mask, updates)
    zeros = jnp.zeros((_CH,), jnp.float32)
    out = _sc_scatter(dest, upd, zeros)
    return _postprocess(out)


# speedup vs baseline: 1.4666x; 1.4666x over previous
import functools

import jax
import jax.numpy as jnp
from jax import lax
from jax.experimental import pallas as pl
from jax.experimental.pallas import tpu as pltpu
from jax.experimental.pallas import tpu_sc as plsc

_B, _H, _W, _C = 4, 112, 112, 192
_WP = 128
_X = 2 * _W
_N = _B * _H * _C * _WP
_NB = _H * _C * _WP
_OB = 2 * _H * _C * _X
_OUT_WORDS = _B * _OB

_TRASH = 16384
_CH = 15 * 131072 - _TRASH
_K = 5
_LAST = _OB - (_K - 1) * _CH
_NSUB = 16
_PT = _NB // _NSUB
_WIN = 1536
_NWIN = _PT // _WIN
_ZSH = _CH // _NSUB
_SSH = _CH // _NSUB
_LSH = _LAST // _NSUB
_NROUND = (_B * _K) // 2


_PRE_BH = 16


def _pre_body(mask_ref, upd_ref, dest_ref, uflat_ref):
    m = mask_ref[...]
    u = upd_ref[...]
    mp = jnp.concatenate(
        [m, jnp.zeros((_PRE_BH, _C, _WP - _W), jnp.int32)], axis=2)
    up = jnp.concatenate(
        [u, jnp.zeros((_PRE_BH, _C, _WP - _W), jnp.float32)], axis=2)
    c = lax.broadcasted_iota(jnp.int32, mp.shape, 1)
    y = mp // (_C * _X)
    x = (mp // _C) % _X
    dest = y * (_C * _X) + c * _X + x
    dest_ref[...] = dest.reshape(_PRE_BH * _C, _WP)
    uflat_ref[...] = up.reshape(_PRE_BH * _C, _WP)


def _preprocess(mask, updates):
    m3 = jnp.transpose(mask, (0, 1, 3, 2)).reshape(_B * _H, _C, _W)
    u3 = jnp.transpose(updates, (0, 1, 3, 2)).reshape(_B * _H, _C, _W)
    nblk = _B * _H // _PRE_BH
    obk = _PRE_BH * _C
    dest, uflat = pl.pallas_call(
        _pre_body,
        out_shape=(jax.ShapeDtypeStruct((_N // _WP, _WP), jnp.int32),
                   jax.ShapeDtypeStruct((_N // _WP, _WP), jnp.float32)),
        grid=(nblk,),
        in_specs=[pl.BlockSpec((_PRE_BH, _C, _W), lambda i: (i, 0, 0)),
                  pl.BlockSpec((_PRE_BH, _C, _W), lambda i: (i, 0, 0))],
        out_specs=(pl.BlockSpec((obk, _WP), lambda i: (i, 0)),
                   pl.BlockSpec((obk, _WP), lambda i: (i, 0))),
    )(m3, u3)
    return dest.reshape(_N), uflat.reshape(_N)


_POST_ROWS = 2048


def _post_body(flat_ref, out_ref):
    a = flat_ref[...].reshape(_POST_ROWS // 4, 7, 128)
    r0 = jnp.concatenate([a[:, 0, :], a[:, 1, :96]], axis=1)
    r1 = jnp.concatenate([a[:, 1, 96:], a[:, 2, :], a[:, 3, :64]], axis=1)
    r2 = jnp.concatenate([a[:, 3, 64:], a[:, 4, :], a[:, 5, :32]], axis=1)
    r3 = jnp.concatenate([a[:, 5, 32:], a[:, 6, :]], axis=1)
    rows = jnp.stack([r0, r1, r2, r3], axis=1)
    out_ref[...] = rows.reshape(_POST_ROWS, _X)


def _postprocess(flat):
    orows = _B * 2 * _H * _C
    ibk = _POST_ROWS * 7 // 4
    out = pl.pallas_call(
        _post_body,
        out_shape=jax.ShapeDtypeStruct((orows, _X), jnp.float32),
        grid=(orows // _POST_ROWS,),
        in_specs=[pl.BlockSpec((ibk, 128), lambda i: (i, 0))],
        out_specs=pl.BlockSpec((_POST_ROWS, _X), lambda i: (i, 0)),
    )(flat.reshape(_OUT_WORDS // 128, 128))
    out = out.reshape(_B, 2 * _H, _C, _X)
    return jnp.transpose(out, (0, 1, 3, 2))


_mesh = plsc.VectorSubcoreMesh(core_axis_name="c", subcore_axis_name="s")


@functools.partial(
    pl.kernel,
    out_type=jax.ShapeDtypeStruct((_OUT_WORDS,), jnp.float32),
    mesh=_mesh,
    scratch_types=[
        pltpu.VMEM_SHARED((_CH + _TRASH,), jnp.float32),
        pltpu.VMEM((_WIN,), jnp.int32),
        pltpu.VMEM((_WIN,), jnp.int32),
        pltpu.VMEM((_WIN,), jnp.float32),
        pltpu.VMEM((_WIN,), jnp.float32),
        pltpu.VMEM((_WIN,), jnp.int32),
        pltpu.SemaphoreType.DMA,
        pltpu.SemaphoreType.DMA,
        pltpu.SemaphoreType.DMA,
        pltpu.SemaphoreType.DMA,
    ],
)
def _sc_scatter(dest_hbm, upd_hbm, zero_hbm, out_hbm, acc_sh, d0, d1, u0, u1,
                idx_v, sd0, sd1, su0, su1):
    core = lax.axis_index("c")
    sub = lax.axis_index("s")

    @pl.loop(0, _NROUND)
    def _(r):
        g = r * 2 + core
        b = g // _K
        k = g % _K
        ibase = b * _NB + sub * _PT
        cbase = k * _CH

        zoff = pl.multiple_of(sub * _ZSH, 8)
        pltpu.sync_copy(zero_hbm.at[pl.ds(zoff, _ZSH)],
                        acc_sh.at[pl.ds(zoff, _ZSH)])
        plsc.subcore_barrier()

        def start_load(w, db, ub, sdb, sub_):
            off = pl.multiple_of(ibase + w * _WIN, 8)
            pltpu.async_copy(dest_hbm.at[pl.ds(off, _WIN)], db, sdb)
            pltpu.async_copy(upd_hbm.at[pl.ds(off, _WIN)], ub, sub_)

        def wait_load(db, ub, sdb, sub_):
            pltpu.make_async_copy(dest_hbm.at[pl.ds(0, _WIN)], db, sdb).wait()
            pltpu.make_async_copy(upd_hbm.at[pl.ds(0, _WIN)], ub, sub_).wait()

        def process(db, ub):
            @pl.loop(0, _WIN // 16)
            def _(i):
                d = db[pl.ds(i * 16, 16)]
                rel = d - cbase
                inb = plsc.bitcast(rel, jnp.uint32) < jnp.uint32(_CH)
                tr = _CH + (d & (_TRASH - 1))
                idx_v[pl.ds(i * 16, 16)] = jnp.where(inb, rel, tr)

            pltpu.sync_copy(ub, acc_sh.at[idx_v], add=True)

        start_load(0, d0, u0, sd0, su0)
        start_load(1, d1, u1, sd1, su1)

        @pl.loop(0, _NWIN // 2 - 1)
        def _(j):
            w = j * 2
            wait_load(d0, u0, sd0, su0)
            process(d0, u0)
            start_load(w + 2, d0, u0, sd0, su0)
            wait_load(d1, u1, sd1, su1)
            process(d1, u1)
            start_load(w + 3, d1, u1, sd1, su1)

        wait_load(d0, u0, sd0, su0)
        process(d0, u0)
        wait_load(d1, u1, sd1, su1)
        process(d1, u1)
        plsc.subcore_barrier()

        @pl.when(k < _K - 1)
        def _():
            soff = pl.multiple_of(sub * _SSH, 8)
            obase = pl.multiple_of(b * _OB + k * _CH + sub * _SSH, 8)
            pltpu.sync_copy(acc_sh.at[pl.ds(soff, _SSH)],
                            out_hbm.at[pl.ds(obase, _SSH)])

        @pl.when(k == _K - 1)
        def _():
            soff = pl.multiple_of(sub * _LSH, 8)
            obase = pl.multiple_of(b * _OB + k * _CH + sub * _LSH, 8)
            pltpu.sync_copy(acc_sh.at[pl.ds(soff, _LSH)],
                            out_hbm.at[pl.ds(obase, _LSH)])

        plsc.subcore_barrier()


def kernel(updates, mask):
    mask = mask.astype(jnp.int32)
    dest, upd = _preprocess(mask, updates)
    zeros = jnp.zeros((_CH,), jnp.float32)
    out = _sc_scatter(dest, upd, zeros)
    return _postprocess(out)


# speedup vs baseline: 1.5810x; 1.0780x over previous
import functools

import jax
import jax.numpy as jnp
from jax import lax
from jax.experimental import pallas as pl
from jax.experimental.pallas import tpu as pltpu
from jax.experimental.pallas import tpu_sc as plsc

_B, _H, _W, _C = 4, 112, 112, 192
_WP = 128
_X = 2 * _W
_N = _B * _H * _C * _WP
_NB = _H * _C * _WP
_OB = 2 * _H * _C * _X
_OUT_WORDS = _B * _OB

_TRASH = 16384
_CH = 15 * 131072 - _TRASH
_K = 5
_LAST = _OB - (_K - 1) * _CH
_NSUB = 16
_PT = _NB // _NSUB
_WIN = 1536
_NWIN = _PT // _WIN
_ZSH = _CH // _NSUB
_SSH = _CH // _NSUB
_LSH = _LAST // _NSUB
_NROUND = (_B * _K) // 2


_PRE_BH = 16


def _pre_body(mask_ref, upd_ref, dest_ref, uflat_ref):
    m = mask_ref[...]
    u = upd_ref[...]
    mp = jnp.concatenate(
        [m, jnp.zeros((_PRE_BH, _C, _WP - _W), jnp.int32)], axis=2)
    up = jnp.concatenate(
        [u, jnp.zeros((_PRE_BH, _C, _WP - _W), jnp.float32)], axis=2)
    c = lax.broadcasted_iota(jnp.int32, mp.shape, 1)
    y = mp // (_C * _X)
    x = (mp // _C) % _X
    dest = y * (_C * _X) + c * _X + x
    dest_ref[...] = dest.reshape(_PRE_BH * _C, _WP)
    uflat_ref[...] = up.reshape(_PRE_BH * _C, _WP)


def _preprocess(mask, updates):
    m3 = jnp.transpose(mask, (0, 1, 3, 2)).reshape(_B * _H, _C, _W)
    u3 = jnp.transpose(updates, (0, 1, 3, 2)).reshape(_B * _H, _C, _W)
    nblk = _B * _H // _PRE_BH
    obk = _PRE_BH * _C
    dest, uflat = pl.pallas_call(
        _pre_body,
        out_shape=(jax.ShapeDtypeStruct((_N // _WP, _WP), jnp.int32),
                   jax.ShapeDtypeStruct((_N // _WP, _WP), jnp.float32)),
        grid=(nblk,),
        in_specs=[pl.BlockSpec((_PRE_BH, _C, _W), lambda i: (i, 0, 0)),
                  pl.BlockSpec((_PRE_BH, _C, _W), lambda i: (i, 0, 0))],
        out_specs=(pl.BlockSpec((obk, _WP), lambda i: (i, 0)),
                   pl.BlockSpec((obk, _WP), lambda i: (i, 0))),
    )(m3, u3)
    return dest.reshape(_N), uflat.reshape(_N)


_POST_ROWS = 2048
_OROWS = _B * 2 * _H * _C
_HROWS = _OROWS // 2


def _post_body(flat_ref, out_ref):
    a = flat_ref[...].reshape(_POST_ROWS // 4, 7, 128)
    r0 = jnp.concatenate([a[:, 0, :], a[:, 1, :96]], axis=1)
    r1 = jnp.concatenate([a[:, 1, 96:], a[:, 2, :], a[:, 3, :64]], axis=1)
    r2 = jnp.concatenate([a[:, 3, 64:], a[:, 4, :], a[:, 5, :32]], axis=1)
    r3 = jnp.concatenate([a[:, 5, 32:], a[:, 6, :]], axis=1)
    rows = jnp.stack([r0, r1, r2, r3], axis=1)
    out_ref[...] = rows.reshape(_POST_ROWS, _X)


def _post_body2(_, flat_ref, out_ref):
    _post_body(flat_ref, out_ref)


def _postprocess(flat_a, flat_b):
    nblk = _HROWS // _POST_ROWS
    ibk = _POST_ROWS * 7 // 4
    half = pl.pallas_call(
        _post_body,
        out_shape=jax.ShapeDtypeStruct((_OROWS, _X), jnp.float32),
        grid=(nblk,),
        in_specs=[pl.BlockSpec((ibk, 128), lambda i: (i, 0))],
        out_specs=pl.BlockSpec((_POST_ROWS, _X), lambda i: (i, 0)),
    )(flat_a.reshape(_OUT_WORDS // 2 // 128, 128))
    out = pl.pallas_call(
        _post_body2,
        out_shape=jax.ShapeDtypeStruct((_OROWS, _X), jnp.float32),
        grid=(nblk,),
        in_specs=[pl.BlockSpec(memory_space=pl.ANY),
                  pl.BlockSpec((ibk, 128), lambda i: (i, 0))],
        out_specs=pl.BlockSpec((_POST_ROWS, _X),
                               lambda i: (i + nblk, 0)),
        input_output_aliases={0: 0},
    )(half, flat_b.reshape(_OUT_WORDS // 2 // 128, 128))
    out = out.reshape(_B, 2 * _H, _C, _X)
    return jnp.transpose(out, (0, 1, 3, 2))


_mesh = plsc.VectorSubcoreMesh(core_axis_name="c", subcore_axis_name="s")


def _make_sc_scatter(bo):
  @functools.partial(
      pl.kernel,
      out_type=jax.ShapeDtypeStruct((_OUT_WORDS // 2,), jnp.float32),
      mesh=_mesh,
      scratch_types=[
          pltpu.VMEM_SHARED((_CH + _TRASH,), jnp.float32),
          pltpu.VMEM((_WIN,), jnp.int32),
          pltpu.VMEM((_WIN,), jnp.int32),
          pltpu.VMEM((_WIN,), jnp.float32),
          pltpu.VMEM((_WIN,), jnp.float32),
          pltpu.VMEM((_WIN,), jnp.int32),
          pltpu.SemaphoreType.DMA,
          pltpu.SemaphoreType.DMA,
          pltpu.SemaphoreType.DMA,
          pltpu.SemaphoreType.DMA,
      ],
  )
  def _sc_scatter(dest_hbm, upd_hbm, zero_hbm, out_hbm, acc_sh, d0, d1, u0,
                  u1, idx_v, sd0, sd1, su0, su1):
    core = lax.axis_index("c")
    sub = lax.axis_index("s")

    @pl.loop(0, _NROUND // 2)
    def _(r):
        g = bo * _K + r * 2 + core
        b = g // _K
        k = g % _K
        ibase = b * _NB + sub * _PT
        cbase = k * _CH

        zoff = pl.multiple_of(sub * _ZSH, 8)
        pltpu.sync_copy(zero_hbm.at[pl.ds(zoff, _ZSH)],
                        acc_sh.at[pl.ds(zoff, _ZSH)])
        plsc.subcore_barrier()

        def start_load(w, db, ub, sdb, sub_):
            off = pl.multiple_of(ibase + w * _WIN, 8)
            pltpu.async_copy(dest_hbm.at[pl.ds(off, _WIN)], db, sdb)
            pltpu.async_copy(upd_hbm.at[pl.ds(off, _WIN)], ub, sub_)

        def wait_load(db, ub, sdb, sub_):
            pltpu.make_async_copy(dest_hbm.at[pl.ds(0, _WIN)], db, sdb).wait()
            pltpu.make_async_copy(upd_hbm.at[pl.ds(0, _WIN)], ub, sub_).wait()

        def process(db, ub):
            @pl.loop(0, _WIN // 16)
            def _(i):
                d = db[pl.ds(i * 16, 16)]
                rel = d - cbase
                inb = plsc.bitcast(rel, jnp.uint32) < jnp.uint32(_CH)
                tr = _CH + (d & (_TRASH - 1))
                idx_v[pl.ds(i * 16, 16)] = jnp.where(inb, rel, tr)

            pltpu.sync_copy(ub, acc_sh.at[idx_v], add=True)

        start_load(0, d0, u0, sd0, su0)
        start_load(1, d1, u1, sd1, su1)

        @pl.loop(0, _NWIN // 2 - 1)
        def _(j):
            w = j * 2
            wait_load(d0, u0, sd0, su0)
            process(d0, u0)
            start_load(w + 2, d0, u0, sd0, su0)
            wait_load(d1, u1, sd1, su1)
            process(d1, u1)
            start_load(w + 3, d1, u1, sd1, su1)

        wait_load(d0, u0, sd0, su0)
        process(d0, u0)
        wait_load(d1, u1, sd1, su1)
        process(d1, u1)
        plsc.subcore_barrier()

        @pl.when(k < _K - 1)
        def _():
            soff = pl.multiple_of(sub * _SSH, 8)
            obase = pl.multiple_of((b - bo) * _OB + k * _CH + sub * _SSH, 8)
            pltpu.sync_copy(acc_sh.at[pl.ds(soff, _SSH)],
                            out_hbm.at[pl.ds(obase, _SSH)])

        @pl.when(k == _K - 1)
        def _():
            soff = pl.multiple_of(sub * _LSH, 8)
            obase = pl.multiple_of((b - bo) * _OB + k * _CH + sub * _LSH, 8)
            pltpu.sync_copy(acc_sh.at[pl.ds(soff, _LSH)],
                            out_hbm.at[pl.ds(obase, _LSH)])

        plsc.subcore_barrier()

  return _sc_scatter


_sc_scatter_a = _make_sc_scatter(0)
_sc_scatter_b = _make_sc_scatter(2)


def kernel(updates, mask):
    mask = mask.astype(jnp.int32)
    dest, upd = _preprocess(mask, updates)
    zeros = jnp.zeros((_CH,), jnp.float32)
    out_a = _sc_scatter_a(dest, upd, zeros)
    out_b = _sc_scatter_b(dest, upd, zeros)
    return _postprocess(out_a, out_b)


# speedup vs baseline: 1.6602x; 1.0501x over previous
import functools

import jax
import jax.numpy as jnp
from jax import lax
from jax.experimental import pallas as pl
from jax.experimental.pallas import tpu as pltpu
from jax.experimental.pallas import tpu_sc as plsc

_B, _H, _W, _C = 4, 112, 112, 192
_X = 2 * _W
_N = _B * _H * _C * _W
_NB = _H * _C * _W
_OB = 2 * _H * _C * _X
_OUT_WORDS = _B * _OB

_TRASH = 16384
_CH = 15 * 131072 - _TRASH
_K = 5
_LAST = _OB - (_K - 1) * _CH
_NSUB = 16
_PT = _NB // _NSUB
_WIN = 1536
_NWIN = _PT // _WIN
_ZSH = _CH // _NSUB
_SSH = _CH // _NSUB
_LSH = _LAST // _NSUB
_NROUND = (_B * _K) // 2


_PRE_BH = 16


def _compact(v):
    r = v.shape[0]
    a = v.reshape(r // 8, 8, _W)
    outs = []
    for i in range(7):
        off = 16 * i
        outs.append(jnp.concatenate(
            [a[:, i, off:], a[:, i + 1, :off + 16]], axis=1))
    return jnp.stack(outs, axis=1).reshape(r * 7 // 8, 128)


def _pre_body(mask_ref, upd_ref, dest_ref, uflat_ref):
    m = mask_ref[...]
    u = upd_ref[...]
    c = lax.broadcasted_iota(jnp.int32, m.shape, 1)
    y = m // (_C * _X)
    x = (m // _C) % _X
    dest = y * (_C * _X) + c * _X + x
    dest_ref[...] = _compact(dest.reshape(_PRE_BH * _C, _W))
    uflat_ref[...] = _compact(u.reshape(_PRE_BH * _C, _W))


def _preprocess(mask, updates, h):
    m3 = jnp.transpose(mask, (0, 1, 3, 2)).reshape(_B * _H, _C, _W)
    u3 = jnp.transpose(updates, (0, 1, 3, 2)).reshape(_B * _H, _C, _W)
    nblk = _B * _H // _PRE_BH // 2
    obk = _PRE_BH * _C * _W // 128
    dest, uflat = pl.pallas_call(
        _pre_body,
        out_shape=(jax.ShapeDtypeStruct((_N // 2 // 128, 128), jnp.int32),
                   jax.ShapeDtypeStruct((_N // 2 // 128, 128), jnp.float32)),
        grid=(nblk,),
        in_specs=[
            pl.BlockSpec((_PRE_BH, _C, _W), lambda i: (i + h * nblk, 0, 0)),
            pl.BlockSpec((_PRE_BH, _C, _W), lambda i: (i + h * nblk, 0, 0))],
        out_specs=(pl.BlockSpec((obk, 128), lambda i: (i, 0)),
                   pl.BlockSpec((obk, 128), lambda i: (i, 0))),
    )(m3, u3)
    return dest.reshape(_N // 2), uflat.reshape(_N // 2)


_POST_ROWS = 2048
_OROWS = _B * 2 * _H * _C
_HROWS = _OROWS // 2


def _post_body(flat_ref, out_ref):
    a = flat_ref[...].reshape(_POST_ROWS // 4, 7, 128)
    r0 = jnp.concatenate([a[:, 0, :], a[:, 1, :96]], axis=1)
    r1 = jnp.concatenate([a[:, 1, 96:], a[:, 2, :], a[:, 3, :64]], axis=1)
    r2 = jnp.concatenate([a[:, 3, 64:], a[:, 4, :], a[:, 5, :32]], axis=1)
    r3 = jnp.concatenate([a[:, 5, 32:], a[:, 6, :]], axis=1)
    rows = jnp.stack([r0, r1, r2, r3], axis=1)
    out_ref[...] = rows.reshape(_POST_ROWS, _X)


def _post_body2(_, flat_ref, out_ref):
    _post_body(flat_ref, out_ref)


def _postprocess(flat_a, flat_b):
    nblk = _HROWS // _POST_ROWS
    ibk = _POST_ROWS * 7 // 4
    half = pl.pallas_call(
        _post_body,
        out_shape=jax.ShapeDtypeStruct((_OROWS, _X), jnp.float32),
        grid=(nblk,),
        in_specs=[pl.BlockSpec((ibk, 128), lambda i: (i, 0))],
        out_specs=pl.BlockSpec((_POST_ROWS, _X), lambda i: (i, 0)),
    )(flat_a.reshape(_OUT_WORDS // 2 // 128, 128))
    out = pl.pallas_call(
        _post_body2,
        out_shape=jax.ShapeDtypeStruct((_OROWS, _X), jnp.float32),
        grid=(nblk,),
        in_specs=[pl.BlockSpec(memory_space=pl.ANY),
                  pl.BlockSpec((ibk, 128), lambda i: (i, 0))],
        out_specs=pl.BlockSpec((_POST_ROWS, _X),
                               lambda i: (i + nblk, 0)),
        input_output_aliases={0: 0},
    )(half, flat_b.reshape(_OUT_WORDS // 2 // 128, 128))
    out = out.reshape(_B, 2 * _H, _C, _X)
    return jnp.transpose(out, (0, 1, 3, 2))


_mesh = plsc.VectorSubcoreMesh(core_axis_name="c", subcore_axis_name="s")


def _make_sc_scatter(bo):
  @functools.partial(
      pl.kernel,
      out_type=jax.ShapeDtypeStruct((_OUT_WORDS // 2,), jnp.float32),
      mesh=_mesh,
      scratch_types=[
          pltpu.VMEM_SHARED((_CH + _TRASH,), jnp.float32),
          pltpu.VMEM((_WIN,), jnp.int32),
          pltpu.VMEM((_WIN,), jnp.int32),
          pltpu.VMEM((_WIN,), jnp.float32),
          pltpu.VMEM((_WIN,), jnp.float32),
          pltpu.VMEM((_WIN,), jnp.int32),
          pltpu.SemaphoreType.DMA,
          pltpu.SemaphoreType.DMA,
          pltpu.SemaphoreType.DMA,
          pltpu.SemaphoreType.DMA,
      ],
  )
  def _sc_scatter(dest_hbm, upd_hbm, zero_hbm, out_hbm, acc_sh, d0, d1, u0,
                  u1, idx_v, sd0, sd1, su0, su1):
    core = lax.axis_index("c")
    sub = lax.axis_index("s")

    @pl.loop(0, _NROUND // 2)
    def _(r):
        g = bo * _K + r * 2 + core
        b = g // _K
        k = g % _K
        ibase = (b - bo) * _NB + sub * _PT
        cbase = k * _CH

        zoff = pl.multiple_of(sub * _ZSH, 8)
        pltpu.sync_copy(zero_hbm.at[pl.ds(zoff, _ZSH)],
                        acc_sh.at[pl.ds(zoff, _ZSH)])
        plsc.subcore_barrier()

        def start_load(w, db, ub, sdb, sub_):
            off = pl.multiple_of(ibase + w * _WIN, 8)
            pltpu.async_copy(dest_hbm.at[pl.ds(off, _WIN)], db, sdb)
            pltpu.async_copy(upd_hbm.at[pl.ds(off, _WIN)], ub, sub_)

        def wait_load(db, ub, sdb, sub_):
            pltpu.make_async_copy(dest_hbm.at[pl.ds(0, _WIN)], db, sdb).wait()
            pltpu.make_async_copy(upd_hbm.at[pl.ds(0, _WIN)], ub, sub_).wait()

        def process(db, ub):
            @pl.loop(0, _WIN // 16)
            def _(i):
                d = db[pl.ds(i * 16, 16)]
                rel = d - cbase
                inb = plsc.bitcast(rel, jnp.uint32) < jnp.uint32(_CH)
                tr = _CH + (d & (_TRASH - 1))
                idx_v[pl.ds(i * 16, 16)] = jnp.where(inb, rel, tr)

            pltpu.sync_copy(ub, acc_sh.at[idx_v], add=True)

        start_load(0, d0, u0, sd0, su0)
        start_load(1, d1, u1, sd1, su1)

        @pl.loop(0, _NWIN // 2 - 1)
        def _(j):
            w = j * 2
            wait_load(d0, u0, sd0, su0)
            process(d0, u0)
            start_load(w + 2, d0, u0, sd0, su0)
            wait_load(d1, u1, sd1, su1)
            process(d1, u1)
            start_load(w + 3, d1, u1, sd1, su1)

        wait_load(d0, u0, sd0, su0)
        process(d0, u0)
        wait_load(d1, u1, sd1, su1)
        process(d1, u1)
        plsc.subcore_barrier()

        @pl.when(k < _K - 1)
        def _():
            soff = pl.multiple_of(sub * _SSH, 8)
            obase = pl.multiple_of((b - bo) * _OB + k * _CH + sub * _SSH, 8)
            pltpu.sync_copy(acc_sh.at[pl.ds(soff, _SSH)],
                            out_hbm.at[pl.ds(obase, _SSH)])

        @pl.when(k == _K - 1)
        def _():
            soff = pl.multiple_of(sub * _LSH, 8)
            obase = pl.multiple_of((b - bo) * _OB + k * _CH + sub * _LSH, 8)
            pltpu.sync_copy(acc_sh.at[pl.ds(soff, _LSH)],
                            out_hbm.at[pl.ds(obase, _LSH)])

        plsc.subcore_barrier()

  return _sc_scatter


_sc_scatter_a = _make_sc_scatter(0)
_sc_scatter_b = _make_sc_scatter(2)


def kernel(updates, mask):
    mask = mask.astype(jnp.int32)
    dest_a, upd_a = _preprocess(mask, updates, 0)
    dest_b, upd_b = _preprocess(mask, updates, 1)
    zeros = jnp.zeros((_CH,), jnp.float32)
    out_a = _sc_scatter_a(dest_a, upd_a, zeros)
    out_b = _sc_scatter_b(dest_b, upd_b, zeros)
    return _postprocess(out_a, out_b)


# speedup vs baseline: 1.7109x; 1.0305x over previous
import functools

import jax
import jax.numpy as jnp
from jax import lax
from jax.experimental import pallas as pl
from jax.experimental.pallas import tpu as pltpu
from jax.experimental.pallas import tpu_sc as plsc

_B, _H, _W, _C = 4, 112, 112, 192
_X = 2 * _W
_N = _B * _H * _C * _W
_NB = _H * _C * _W
_OB = 2 * _H * _C * _X
_OUT_WORDS = _B * _OB

_TRASH = 16384
_CH = 15 * 131072 - _TRASH
_K = 5
_LAST = _OB - (_K - 1) * _CH
_NSUB = 16
_PT = _NB // _NSUB
_WIN = 1536
_NWIN = _PT // _WIN
_ZSH = _CH // _NSUB
_SSH = _CH // _NSUB
_LSH = _LAST // _NSUB
_NROUND = (_B * _K) // 2


_PRE_BH = 16


def _compact(v):
    r = v.shape[0]
    a = v.reshape(r // 8, 8, _W)
    outs = []
    for i in range(7):
        off = 16 * i
        outs.append(jnp.concatenate(
            [a[:, i, off:], a[:, i + 1, :off + 16]], axis=1))
    return jnp.stack(outs, axis=1).reshape(r * 7 // 8, 128)


def _pre_body(mask_ref, upd_ref, dest_ref, uflat_ref):
    m = mask_ref[...]
    u = upd_ref[...]
    c = lax.broadcasted_iota(jnp.int32, m.shape, 1)
    n = (m >> 6).astype(jnp.uint32)
    hi = n >> 16
    t = (n & 0xFFFF) + hi
    r = (21845 * hi + ((t * 43691) >> 17)).astype(jnp.int32)
    y = ((r >> 5) * 9363) >> 16
    x = r - y * _X
    dest = y * (_C * _X) + c * _X + x
    dest_ref[...] = _compact(dest.reshape(_PRE_BH * _C, _W))
    uflat_ref[...] = _compact(u.reshape(_PRE_BH * _C, _W))


def _preprocess(mask, updates, h):
    m3 = jnp.transpose(mask, (0, 1, 3, 2)).reshape(_B * _H, _C, _W)
    u3 = jnp.transpose(updates, (0, 1, 3, 2)).reshape(_B * _H, _C, _W)
    nblk = _B * _H // _PRE_BH // 2
    obk = _PRE_BH * _C * _W // 128
    dest, uflat = pl.pallas_call(
        _pre_body,
        out_shape=(jax.ShapeDtypeStruct((_N // 2 // 128, 128), jnp.int32),
                   jax.ShapeDtypeStruct((_N // 2 // 128, 128), jnp.float32)),
        grid=(nblk,),
        in_specs=[
            pl.BlockSpec((_PRE_BH, _C, _W), lambda i: (i + h * nblk, 0, 0)),
            pl.BlockSpec((_PRE_BH, _C, _W), lambda i: (i + h * nblk, 0, 0))],
        out_specs=(pl.BlockSpec((obk, 128), lambda i: (i, 0)),
                   pl.BlockSpec((obk, 128), lambda i: (i, 0))),
    )(m3, u3)
    return dest.reshape(_N // 2), uflat.reshape(_N // 2)


_POST_ROWS = 2048
_OROWS = _B * 2 * _H * _C
_HROWS = _OROWS // 2


def _post_body(flat_ref, out_ref):
    a = flat_ref[...].reshape(_POST_ROWS // 4, 7, 128)
    r0 = jnp.concatenate([a[:, 0, :], a[:, 1, :96]], axis=1)
    r1 = jnp.concatenate([a[:, 1, 96:], a[:, 2, :], a[:, 3, :64]], axis=1)
    r2 = jnp.concatenate([a[:, 3, 64:], a[:, 4, :], a[:, 5, :32]], axis=1)
    r3 = jnp.concatenate([a[:, 5, 32:], a[:, 6, :]], axis=1)
    rows = jnp.stack([r0, r1, r2, r3], axis=1)
    out_ref[...] = rows.reshape(_POST_ROWS, _X)


def _post_body2(_, flat_ref, out_ref):
    _post_body(flat_ref, out_ref)


def _postprocess(flat_a, flat_b):
    nblk = _HROWS // _POST_ROWS
    ibk = _POST_ROWS * 7 // 4
    half = pl.pallas_call(
        _post_body,
        out_shape=jax.ShapeDtypeStruct((_OROWS, _X), jnp.float32),
        grid=(nblk,),
        in_specs=[pl.BlockSpec((ibk, 128), lambda i: (i, 0))],
        out_specs=pl.BlockSpec((_POST_ROWS, _X), lambda i: (i, 0)),
    )(flat_a.reshape(_OUT_WORDS // 2 // 128, 128))
    out = pl.pallas_call(
        _post_body2,
        out_shape=jax.ShapeDtypeStruct((_OROWS, _X), jnp.float32),
        grid=(nblk,),
        in_specs=[pl.BlockSpec(memory_space=pl.ANY),
                  pl.BlockSpec((ibk, 128), lambda i: (i, 0))],
        out_specs=pl.BlockSpec((_POST_ROWS, _X),
                               lambda i: (i + nblk, 0)),
        input_output_aliases={0: 0},
    )(half, flat_b.reshape(_OUT_WORDS // 2 // 128, 128))
    out = out.reshape(_B, 2 * _H, _C, _X)
    return jnp.transpose(out, (0, 1, 3, 2))


_mesh = plsc.VectorSubcoreMesh(core_axis_name="c", subcore_axis_name="s")


def _make_sc_scatter(bo):
  @functools.partial(
      pl.kernel,
      out_type=jax.ShapeDtypeStruct((_OUT_WORDS // 2,), jnp.float32),
      mesh=_mesh,
      scratch_types=[
          pltpu.VMEM_SHARED((_CH + _TRASH,), jnp.float32),
          pltpu.VMEM((_WIN,), jnp.int32),
          pltpu.VMEM((_WIN,), jnp.int32),
          pltpu.VMEM((_WIN,), jnp.float32),
          pltpu.VMEM((_WIN,), jnp.float32),
          pltpu.VMEM((_WIN,), jnp.int32),
          pltpu.SemaphoreType.DMA,
          pltpu.SemaphoreType.DMA,
          pltpu.SemaphoreType.DMA,
          pltpu.SemaphoreType.DMA,
      ],
  )
  def _sc_scatter(dest_hbm, upd_hbm, zero_hbm, out_hbm, acc_sh, d0, d1, u0,
                  u1, idx_v, sd0, sd1, su0, su1):
    core = lax.axis_index("c")
    sub = lax.axis_index("s")

    @pl.loop(0, _NROUND // 2)
    def _(r):
        g = bo * _K + r * 2 + core
        b = g // _K
        k = g % _K
        ibase = (b - bo) * _NB + sub * _PT
        cbase = k * _CH

        zoff = pl.multiple_of(sub * _ZSH, 8)
        pltpu.sync_copy(zero_hbm.at[pl.ds(zoff, _ZSH)],
                        acc_sh.at[pl.ds(zoff, _ZSH)])
        plsc.subcore_barrier()

        def start_load(w, db, ub, sdb, sub_):
            off = pl.multiple_of(ibase + w * _WIN, 8)
            pltpu.async_copy(dest_hbm.at[pl.ds(off, _WIN)], db, sdb)
            pltpu.async_copy(upd_hbm.at[pl.ds(off, _WIN)], ub, sub_)

        def wait_load(db, ub, sdb, sub_):
            pltpu.make_async_copy(dest_hbm.at[pl.ds(0, _WIN)], db, sdb).wait()
            pltpu.make_async_copy(upd_hbm.at[pl.ds(0, _WIN)], ub, sub_).wait()

        def process(db, ub):
            @pl.loop(0, _WIN // 16)
            def _(i):
                d = db[pl.ds(i * 16, 16)]
                rel = d - cbase
                inb = plsc.bitcast(rel, jnp.uint32) < jnp.uint32(_CH)
                tr = _CH + (d & (_TRASH - 1))
                idx_v[pl.ds(i * 16, 16)] = jnp.where(inb, rel, tr)

            pltpu.sync_copy(ub, acc_sh.at[idx_v], add=True)

        start_load(0, d0, u0, sd0, su0)
        start_load(1, d1, u1, sd1, su1)

        @pl.loop(0, _NWIN // 2 - 1)
        def _(j):
            w = j * 2
            wait_load(d0, u0, sd0, su0)
            process(d0, u0)
            start_load(w + 2, d0, u0, sd0, su0)
            wait_load(d1, u1, sd1, su1)
            process(d1, u1)
            start_load(w + 3, d1, u1, sd1, su1)

        wait_load(d0, u0, sd0, su0)
        process(d0, u0)
        wait_load(d1, u1, sd1, su1)
        process(d1, u1)
        plsc.subcore_barrier()

        @pl.when(k < _K - 1)
        def _():
            soff = pl.multiple_of(sub * _SSH, 8)
            obase = pl.multiple_of((b - bo) * _OB + k * _CH + sub * _SSH, 8)
            pltpu.sync_copy(acc_sh.at[pl.ds(soff, _SSH)],
                            out_hbm.at[pl.ds(obase, _SSH)])

        @pl.when(k == _K - 1)
        def _():
            soff = pl.multiple_of(sub * _LSH, 8)
            obase = pl.multiple_of((b - bo) * _OB + k * _CH + sub * _LSH, 8)
            pltpu.sync_copy(acc_sh.at[pl.ds(soff, _LSH)],
                            out_hbm.at[pl.ds(obase, _LSH)])

        plsc.subcore_barrier()

  return _sc_scatter


_sc_scatter_a = _make_sc_scatter(0)
_sc_scatter_b = _make_sc_scatter(2)


def kernel(updates, mask):
    mask = mask.astype(jnp.int32)
    dest_a, upd_a = _preprocess(mask, updates, 0)
    dest_b, upd_b = _preprocess(mask, updates, 1)
    zeros = jnp.zeros((_CH,), jnp.float32)
    out_a = _sc_scatter_a(dest_a, upd_a, zeros)
    out_b = _sc_scatter_b(dest_b, upd_b, zeros)
    return _postprocess(out_a, out_b)
